# Initial kernel scaffold; baseline (speedup 1.0000x reference)
import jax
import jax.numpy as jnp
from jax.experimental import pallas as pl


def kernel(x, edge_index, batch, Wr1, br1, Wroot1, se1_w1, se1_w2, Wr2, br2, Wroot2, se2_w1, se2_w2, Wr3, br3, Wroot3, se3_w1, se3_w2, C1w, C1b, C2w, C2b, C3w, C3b, C4w, C4b):
    raise NotImplementedError("write your pallas kernel here")


# speedup vs baseline: 4.1105x; 4.1105x over previous
import functools

import jax
import jax.numpy as jnp
from jax import lax
from jax.experimental import pallas as pl
from jax.experimental.pallas import tpu as pltpu
from jax.experimental.pallas import tpu_sc as plsc

N = 10000
E = 320000
G = 64
NC = 2
NS = 16
K = 80
NPAD = 10240
RPS = NPAD // NS
ZR = 32
R = 1000
NBLK = N // R

_f32 = jnp.float32


def _seg_sum_sc(h_tab, srcs, dst, split_cols):
    eps = (E // NS) if split_cols else (E // (NC * NS))
    niter = eps // K
    mesh = plsc.VectorSubcoreMesh(
        core_axis_name="c", subcore_axis_name="s", num_cores=NC, num_subcores=NS
    )

    @functools.partial(
        pl.kernel,
        out_type=jax.ShapeDtypeStruct((NC, NPAD, 128), _f32),
        mesh=mesh,
        scratch_types=[
            pltpu.VMEM((K,), jnp.int32),
            pltpu.VMEM((K,), jnp.int32),
            pltpu.VMEM((K, 128), _f32),
            pltpu.VMEM((ZR, 128), _f32),
            pltpu.VMEM_SHARED((NPAD, 128), _f32),
            pltpu.SemaphoreType.DMA,
        ],
    )
    def k(h_hbm, srcs_hbm, dst_hbm, out_hbm, src_v, dst_v, rows_v, zbuf, acc, sem):
        c = lax.axis_index("c")
        s = lax.axis_index("s")
        zv = jnp.zeros((16,), _f32)
        for r in range(ZR):
            for j in range(128 // 16):
                zbuf[r, pl.ds(j * 16, 16)] = zv
        row0 = s * RPS
        for i in range(RPS // ZR):
            pltpu.sync_copy(zbuf, acc.at[pl.ds(row0 + i * ZR, ZR)])
        plsc.subcore_barrier()

        ebase = (s * eps) if split_cols else ((c * NS + s) * eps)

        def body(i, carry):
            e0 = ebase + i * K
            pltpu.sync_copy(srcs_hbm.at[pl.ds(c * E + e0, K)], src_v)
            pltpu.sync_copy(dst_hbm.at[pl.ds(e0, K)], dst_v)
            pltpu.async_copy(h_hbm.at[src_v], rows_v, sem).wait()
            pltpu.sync_copy(rows_v, acc.at[dst_v], add=True)
            return carry

        lax.fori_loop(0, niter, body, 0)
        plsc.subcore_barrier()
        pltpu.sync_copy(acc.at[pl.ds(row0, RPS)], out_hbm.at[c, pl.ds(row0, RPS)])

    return k(h_tab, srcs, dst)


def _onehot(batch_blk):
    iota = lax.broadcasted_iota(jnp.int32, (R, G), 1)
    return (iota == batch_blk).astype(_f32)


def _dot(a, b):
    return jnp.dot(a, b, preferred_element_type=_f32,
                   precision=lax.Precision.DEFAULT)


def _dot_exact(a, b):
    return jnp.dot(a, b, preferred_element_type=_f32,
                   precision=lax.Precision.HIGHEST)


def _dotT(a, b):
    return lax.dot_general(a, b, (((0,), (0,)), ((), ())),
                           preferred_element_type=_f32,
                           precision=lax.Precision.HIGHEST)


def _tc_combine(parts, h, WrT, WrootT, br, batch2, dout, concat, h_split,
                with_counts):
    din_a = 256 if concat else 128
    din_h = 256 if h_split else 128

    def body(*refs):
        if with_counts:
            (p_ref, h_ref, wrt_ref, wroott_ref, br_ref, b_ref,
             out_ref, gs_ref, cnt_ref) = refs
        else:
            (p_ref, h_ref, wrt_ref, wroott_ref, br_ref, b_ref,
             out_ref, gs_ref) = refs
        i = pl.program_id(0)
        p = p_ref[...]
        wrt = wrt_ref[...]
        if concat:
            agg_mm = _dot(p[0], wrt[:128]) + _dot(p[1], wrt[128:])
        else:
            agg_mm = _dot(p[0] + p[1], wrt)
        wroott = wroott_ref[...]
        if h_split:
            hv = h_ref[...]
            h_mm = _dot(hv[0], wroott[:128]) + _dot(hv[1], wroott[128:])
        else:
            h_mm = _dot(h_ref[...], wroott)
        hp = jnp.maximum(agg_mm + h_mm + br_ref[...], 0.0)
        out_ref[...] = hp
        oh = _onehot(b_ref[...])

        @pl.when(i == 0)
        def _():
            gs_ref[...] = jnp.zeros_like(gs_ref)
            if with_counts:
                cnt_ref[...] = jnp.zeros_like(cnt_ref)

        gs_ref[...] += _dotT(oh, hp)
        if with_counts:
            cnt_ref[...] += _dotT(oh, jnp.ones((R, 1), _f32))

    out_shapes = [
        jax.ShapeDtypeStruct((N, dout), _f32),
        jax.ShapeDtypeStruct((G, dout), _f32),
    ]
    out_specs = [
        pl.BlockSpec((R, dout), lambda i: (i, 0)),
        pl.BlockSpec((G, dout), lambda i: (0, 0)),
    ]
    if with_counts:
        out_shapes.append(jax.ShapeDtypeStruct((G, 1), _f32))
        out_specs.append(pl.BlockSpec((G, 1), lambda i: (0, 0)))

    h_spec = (pl.BlockSpec((2, R, 128), lambda i: (0, i, 0)) if h_split
              else pl.BlockSpec((R, 128), lambda i: (i, 0)))
    return pl.pallas_call(
        body,
        grid=(NBLK,),
        in_specs=[
            pl.BlockSpec((2, R, 128), lambda i: (0, i, 0)),
            h_spec,
            pl.BlockSpec((din_a, dout), lambda i: (0, 0)),
            pl.BlockSpec((din_h, dout), lambda i: (0, 0)),
            pl.BlockSpec((1, dout), lambda i: (0, 0)),
            pl.BlockSpec((R, 1), lambda i: (i, 0)),
        ],
        out_specs=out_specs,
        out_shape=out_shapes,
    )(parts, h, WrT, WrootT, br, batch2)


def _tc_gate(h_pre, gs, counts, w1T, w2T, batch2, dout, mode):
    d16 = w1T.shape[1]

    def body(hp_ref, gs_ref, cnt_ref, w1_ref, w2_ref, b_ref, out_ref):
        i = pl.program_id(0)
        cnt = jnp.maximum(cnt_ref[...], 1.0)
        mean = gs_ref[...] / cnt
        z = jnp.maximum(_dot(mean, w1_ref[...]), 0.0)
        zz = _dot(z, w2_ref[...])
        gate = 1.0 / (1.0 + jnp.exp(-zz))
        oh = _onehot(b_ref[...])
        grows = _dot_exact(oh, gate)
        out = hp_ref[...] * grows
        if mode == "plain":
            out_ref[...] = out
        elif mode == "split":
            out_ref[0] = out[:, :128]
            out_ref[1] = out[:, 128:]
        else:
            @pl.when(i == 0)
            def _():
                out_ref[...] = jnp.zeros_like(out_ref)

            out_ref[...] += _dotT(oh, out)

    if mode == "plain":
        out_shape = jax.ShapeDtypeStruct((N, dout), _f32)
        out_spec = pl.BlockSpec((R, dout), lambda i: (i, 0))
    elif mode == "split":
        out_shape = jax.ShapeDtypeStruct((2, N, 128), _f32)
        out_spec = pl.BlockSpec((2, R, 128), lambda i: (0, i, 0))
    else:
        out_shape = jax.ShapeDtypeStruct((G, dout), _f32)
        out_spec = pl.BlockSpec((G, dout), lambda i: (0, 0))

    return pl.pallas_call(
        body,
        grid=(NBLK,),
        in_specs=[
            pl.BlockSpec((R, dout), lambda i: (i, 0)),
            pl.BlockSpec((G, dout), lambda i: (0, 0)),
            pl.BlockSpec((G, 1), lambda i: (0, 0)),
            pl.BlockSpec((dout, d16), lambda i: (0, 0)),
            pl.BlockSpec((d16, dout), lambda i: (0, 0)),
            pl.BlockSpec((R, 1), lambda i: (i, 0)),
        ],
        out_specs=out_spec,
        out_shape=out_shape,
    )(h_pre, gs, counts, w1T, w2T, batch2)


def _tc_head(gsf, counts, C1wT, C1b, C2wT, C2b, C3wT, C3b, C4wT, C4b):
    def body(gs_ref, cnt_ref, w1_ref, b1_ref, w2_ref, b2_ref, w3_ref, b3_ref,
             w4_ref, b4_ref, out_ref):
        cnt = jnp.maximum(cnt_ref[...], 1.0)
        g = gs_ref[...] / cnt
        g = jnp.maximum(_dot(g, w1_ref[...]) + b1_ref[...], 0.0)
        g = jnp.maximum(_dot(g, w2_ref[...]) + b2_ref[...], 0.0)
        g = jnp.maximum(_dot(g, w3_ref[...]) + b3_ref[...], 0.0)
        out_ref[...] = _dot(g, w4_ref[...]) + b4_ref[...]

    return pl.pallas_call(
        body,
        out_shape=jax.ShapeDtypeStruct((G, 10), _f32),
    )(gsf, counts, C1wT, C1b.reshape(1, -1), C2wT, C2b.reshape(1, -1),
      C3wT, C3b.reshape(1, -1), C4wT, C4b.reshape(1, -1))


def kernel(x, edge_index, batch, Wr1, br1, Wroot1, se1_w1, se1_w2, Wr2, br2,
           Wroot2, se2_w1, se2_w2, Wr3, br3, Wroot3, se3_w1, se3_w2, C1w, C1b,
           C2w, C2b, C3w, C3b, C4w, C4b):
    src, dst = edge_index[0], edge_index[1]
    srcs_a = jnp.concatenate([src, src])
    srcs_b = jnp.concatenate([src, src + N])
    batch2 = batch.reshape(N, 1)

    p1 = _seg_sum_sc(x, srcs_a, dst, split_cols=False)
    hp1, gs1, counts = _tc_combine(p1, x, Wr1.T, Wroot1.T, br1.reshape(1, -1),
                                   batch2, 128, concat=False, h_split=False,
                                   with_counts=True)
    h1 = _tc_gate(hp1, gs1, counts, se1_w1.T, se1_w2.T, batch2, 128, "plain")

    p2 = _seg_sum_sc(h1, srcs_a, dst, split_cols=False)
    hp2, gs2 = _tc_combine(p2, h1, Wr2.T, Wroot2.T, br2.reshape(1, -1),
                           batch2, 256, concat=False, h_split=False,
                           with_counts=False)
    h2s = _tc_gate(hp2, gs2, counts, se2_w1.T, se2_w2.T, batch2, 256, "split")

    p3 = _seg_sum_sc(h2s.reshape(2 * N, 128), srcs_b, dst, split_cols=True)
    hp3, gs3 = _tc_combine(p3, h2s, Wr3.T, Wroot3.T, br3.reshape(1, -1),
                           batch2, 512, concat=True, h_split=True,
                           with_counts=False)
    gsf = _tc_gate(hp3, gs3, counts, se3_w1.T, se3_w2.T, batch2, 512, "pool")

    return _tc_head(gsf, counts, C1w.T, C1b, C2w.T, C2b, C3w.T, C3b,
                    C4w.T, C4b)


# speedup vs baseline: 6.4686x; 1.5737x over previous
import functools

import jax
import jax.numpy as jnp
from jax import lax
from jax.experimental import pallas as pl
from jax.experimental.pallas import tpu as pltpu
from jax.experimental.pallas import tpu_sc as plsc

N = 10000
E = 320000
G = 64
NC = 2
NS = 16
K = 80
NPAD = 10240
RPS = NPAD // NS
ZR = 32
R = 1000
NBLK = N // R

_f32 = jnp.float32


def _seg_sum_sc(h_tab, src_flat, dst_flat, eps):
    niter = eps // K
    mesh = plsc.VectorSubcoreMesh(
        core_axis_name="c", subcore_axis_name="s", num_cores=NC, num_subcores=NS
    )

    @functools.partial(
        pl.kernel,
        out_type=jax.ShapeDtypeStruct((NC, NPAD, 128), _f32),
        mesh=mesh,
        scratch_types=[
            pltpu.VMEM((K,), jnp.int32),
            pltpu.VMEM((K,), jnp.int32),
            pltpu.VMEM((K,), jnp.int32),
            pltpu.VMEM((K,), jnp.int32),
            pltpu.VMEM((K, 128), _f32),
            pltpu.VMEM((K, 128), _f32),
            pltpu.VMEM((ZR, 128), _f32),
            pltpu.VMEM_SHARED((NPAD, 128), _f32),
            pltpu.SemaphoreType.DMA,
            pltpu.SemaphoreType.DMA,
        ],
    )
    def k(h_hbm, src_hbm, dst_hbm, out_hbm, src0, dst0, src1, dst1,
          rows0, rows1, zbuf, acc, gsem0, gsem1):
        c = lax.axis_index("c")
        s = lax.axis_index("s")
        w = c * NS + s
        ebase = w * eps
        zv = jnp.zeros((16,), _f32)
        for r in range(ZR):
            for j in range(128 // 16):
                zbuf[r, pl.ds(j * 16, 16)] = zv
        row0 = s * RPS
        for i in range(RPS // ZR):
            pltpu.sync_copy(zbuf, acc.at[pl.ds(row0 + i * ZR, ZR)])
        plsc.subcore_barrier()

        def idx(j, sbuf, dbuf):
            pltpu.sync_copy(src_hbm.at[pl.ds(ebase + j * K, K)], sbuf)
            pltpu.sync_copy(dst_hbm.at[pl.ds(ebase + j * K, K)], dbuf)

        def gat(sbuf, rbuf, sem):
            pltpu.async_copy(h_hbm.at[sbuf], rbuf, sem)

        def gwait(rbuf, sem):
            pltpu.make_async_copy(h_hbm.at[src0], rbuf, sem).wait()

        def scat(dbuf, rbuf):
            pltpu.sync_copy(rbuf, acc.at[dbuf], add=True)

        idx(0, src0, dst0)
        gat(src0, rows0, gsem0)

        def pair(j2, carry):
            j = j2 * 2
            idx(j + 1, src1, dst1)
            gat(src1, rows1, gsem1)
            gwait(rows0, gsem0)
            scat(dst0, rows0)
            idx(j + 2, src0, dst0)
            gat(src0, rows0, gsem0)
            gwait(rows1, gsem1)
            scat(dst1, rows1)
            return carry

        if niter % 2 == 1:
            lax.fori_loop(0, (niter - 1) // 2, pair, 0)
            gwait(rows0, gsem0)
            scat(dst0, rows0)
        else:
            lax.fori_loop(0, niter // 2 - 1, pair, 0)
            idx(niter - 1, src1, dst1)
            gat(src1, rows1, gsem1)
            gwait(rows0, gsem0)
            scat(dst0, rows0)
            gwait(rows1, gsem1)
            scat(dst1, rows1)

        plsc.subcore_barrier()
        pltpu.sync_copy(acc.at[pl.ds(row0, RPS)], out_hbm.at[c, pl.ds(row0, RPS)])

    return k(h_tab, src_flat, dst_flat)


def _onehot(batch_blk):
    iota = lax.broadcasted_iota(jnp.int32, (R, G), 1)
    return (iota == batch_blk).astype(_f32)


def _dot(a, b):
    return jnp.dot(a, b, preferred_element_type=_f32,
                   precision=lax.Precision.DEFAULT)


def _dot_exact(a, b):
    return jnp.dot(a, b, preferred_element_type=_f32,
                   precision=lax.Precision.HIGHEST)


def _dotT(a, b):
    return lax.dot_general(a, b, (((0,), (0,)), ((), ())),
                           preferred_element_type=_f32,
                           precision=lax.Precision.HIGHEST)


def _tc_combine(parts, h, WrT, WrootT, br, batch2, dout, concat, h_split,
                with_counts):
    din_a = 256 if concat else 128
    din_h = 256 if h_split else 128

    def body(*refs):
        if with_counts:
            (p_ref, h_ref, wrt_ref, wroott_ref, br_ref, b_ref,
             out_ref, gs_ref, cnt_ref) = refs
        else:
            (p_ref, h_ref, wrt_ref, wroott_ref, br_ref, b_ref,
             out_ref, gs_ref) = refs
        i = pl.program_id(0)
        p = p_ref[...]
        wrt = wrt_ref[...]
        if concat:
            agg_mm = _dot(p[0], wrt[:128]) + _dot(p[1], wrt[128:])
        else:
            agg_mm = _dot(p[0] + p[1], wrt)
        wroott = wroott_ref[...]
        if h_split:
            hv = h_ref[...]
            h_mm = _dot(hv[0], wroott[:128]) + _dot(hv[1], wroott[128:])
        else:
            h_mm = _dot(h_ref[...], wroott)
        hp = jnp.maximum(agg_mm + h_mm + br_ref[...], 0.0)
        out_ref[...] = hp
        oh = _onehot(b_ref[...])

        @pl.when(i == 0)
        def _():
            gs_ref[...] = jnp.zeros_like(gs_ref)
            if with_counts:
                cnt_ref[...] = jnp.zeros_like(cnt_ref)

        gs_ref[...] += _dotT(oh, hp)
        if with_counts:
            cnt_ref[...] += _dotT(oh, jnp.ones((R, 1), _f32))

    out_shapes = [
        jax.ShapeDtypeStruct((N, dout), _f32),
        jax.ShapeDtypeStruct((G, dout), _f32),
    ]
    out_specs = [
        pl.BlockSpec((R, dout), lambda i: (i, 0)),
        pl.BlockSpec((G, dout), lambda i: (0, 0)),
    ]
    if with_counts:
        out_shapes.append(jax.ShapeDtypeStruct((G, 1), _f32))
        out_specs.append(pl.BlockSpec((G, 1), lambda i: (0, 0)))

    h_spec = (pl.BlockSpec((2, R, 128), lambda i: (0, i, 0)) if h_split
              else pl.BlockSpec((R, 128), lambda i: (i, 0)))
    return pl.pallas_call(
        body,
        grid=(NBLK,),
        in_specs=[
            pl.BlockSpec((2, R, 128), lambda i: (0, i, 0)),
            h_spec,
            pl.BlockSpec((din_a, dout), lambda i: (0, 0)),
            pl.BlockSpec((din_h, dout), lambda i: (0, 0)),
            pl.BlockSpec((1, dout), lambda i: (0, 0)),
            pl.BlockSpec((R, 1), lambda i: (i, 0)),
        ],
        out_specs=out_specs,
        out_shape=out_shapes,
    )(parts, h, WrT, WrootT, br, batch2)


def _tc_gate(h_pre, gs, counts, w1T, w2T, batch2, dout, mode):
    d16 = w1T.shape[1]

    def body(hp_ref, gs_ref, cnt_ref, w1_ref, w2_ref, b_ref, out_ref):
        i = pl.program_id(0)
        cnt = jnp.maximum(cnt_ref[...], 1.0)
        mean = gs_ref[...] / cnt
        z = jnp.maximum(_dot(mean, w1_ref[...]), 0.0)
        zz = _dot(z, w2_ref[...])
        gate = 1.0 / (1.0 + jnp.exp(-zz))
        oh = _onehot(b_ref[...])
        grows = _dot_exact(oh, gate)
        out = hp_ref[...] * grows
        if mode == "plain":
            out_ref[...] = out
        elif mode == "split":
            out_ref[0] = out[:, :128]
            out_ref[1] = out[:, 128:]
        else:
            @pl.when(i == 0)
            def _():
                out_ref[...] = jnp.zeros_like(out_ref)

            out_ref[...] += _dotT(oh, out)

    if mode == "plain":
        out_shape = jax.ShapeDtypeStruct((N, dout), _f32)
        out_spec = pl.BlockSpec((R, dout), lambda i: (i, 0))
    elif mode == "split":
        out_shape = jax.ShapeDtypeStruct((2, N, 128), _f32)
        out_spec = pl.BlockSpec((2, R, 128), lambda i: (0, i, 0))
    else:
        out_shape = jax.ShapeDtypeStruct((G, dout), _f32)
        out_spec = pl.BlockSpec((G, dout), lambda i: (0, 0))

    return pl.pallas_call(
        body,
        grid=(NBLK,),
        in_specs=[
            pl.BlockSpec((R, dout), lambda i: (i, 0)),
            pl.BlockSpec((G, dout), lambda i: (0, 0)),
            pl.BlockSpec((G, 1), lambda i: (0, 0)),
            pl.BlockSpec((dout, d16), lambda i: (0, 0)),
            pl.BlockSpec((d16, dout), lambda i: (0, 0)),
            pl.BlockSpec((R, 1), lambda i: (i, 0)),
        ],
        out_specs=out_spec,
        out_shape=out_shape,
    )(h_pre, gs, counts, w1T, w2T, batch2)


def _tc_head(gsf, counts, C1wT, C1b, C2wT, C2b, C3wT, C3b, C4wT, C4b):
    def body(gs_ref, cnt_ref, w1_ref, b1_ref, w2_ref, b2_ref, w3_ref, b3_ref,
             w4_ref, b4_ref, out_ref):
        cnt = jnp.maximum(cnt_ref[...], 1.0)
        g = gs_ref[...] / cnt
        g = jnp.maximum(_dot(g, w1_ref[...]) + b1_ref[...], 0.0)
        g = jnp.maximum(_dot(g, w2_ref[...]) + b2_ref[...], 0.0)
        g = jnp.maximum(_dot(g, w3_ref[...]) + b3_ref[...], 0.0)
        out_ref[...] = _dot(g, w4_ref[...]) + b4_ref[...]

    return pl.pallas_call(
        body,
        out_shape=jax.ShapeDtypeStruct((G, 10), _f32),
    )(gsf, counts, C1wT, C1b.reshape(1, -1), C2wT, C2b.reshape(1, -1),
      C3wT, C3b.reshape(1, -1), C4wT, C4b.reshape(1, -1))


def kernel(x, edge_index, batch, Wr1, br1, Wroot1, se1_w1, se1_w2, Wr2, br2,
           Wroot2, se2_w1, se2_w2, Wr3, br3, Wroot3, se3_w1, se3_w2, C1w, C1b,
           C2w, C2b, C3w, C3b, C4w, C4b):
    src, dst = edge_index[0], edge_index[1]
    eps_a = E // (NC * NS)
    eps_b = E // NS
    src_b = jnp.concatenate([src, src + N])
    dst_b = jnp.concatenate([dst, dst])
    batch2 = batch.reshape(N, 1)

    p1 = _seg_sum_sc(x, src, dst, eps_a)
    hp1, gs1, counts = _tc_combine(p1, x, Wr1.T, Wroot1.T, br1.reshape(1, -1),
                                   batch2, 128, concat=False, h_split=False,
                                   with_counts=True)
    h1 = _tc_gate(hp1, gs1, counts, se1_w1.T, se1_w2.T, batch2, 128, "plain")

    p2 = _seg_sum_sc(h1, src, dst, eps_a)
    hp2, gs2 = _tc_combine(p2, h1, Wr2.T, Wroot2.T, br2.reshape(1, -1),
                           batch2, 256, concat=False, h_split=False,
                           with_counts=False)
    h2s = _tc_gate(hp2, gs2, counts, se2_w1.T, se2_w2.T, batch2, 256, "split")

    p3 = _seg_sum_sc(h2s.reshape(2 * N, 128), src_b, dst_b, eps_b)
    hp3, gs3 = _tc_combine(p3, h2s, Wr3.T, Wroot3.T, br3.reshape(1, -1),
                           batch2, 512, concat=True, h_split=True,
                           with_counts=False)
    gsf = _tc_gate(hp3, gs3, counts, se3_w1.T, se3_w2.T, batch2, 512, "pool")

    return _tc_head(gsf, counts, C1w.T, C1b, C2w.T, C2b, C3w.T, C3b,
                    C4w.T, C4b)


# speedup vs baseline: 6.4797x; 1.0017x over previous
import functools

import jax
import jax.numpy as jnp
from jax import lax
from jax.experimental import pallas as pl
from jax.experimental.pallas import tpu as pltpu
from jax.experimental.pallas import tpu_sc as plsc

N = 10000
E = 320000
G = 64
NC = 2
NS = 16
K = 80
NBUF = 4
NPAD = 10240
RPS = NPAD // NS
ZR = 32
R = 1000
NBLK = N // R

_f32 = jnp.float32


def _seg_sum_sc(h_tab, src_flat, dst_flat, eps):
    niter = eps // K
    mesh = plsc.VectorSubcoreMesh(
        core_axis_name="c", subcore_axis_name="s", num_cores=NC, num_subcores=NS
    )

    @functools.partial(
        pl.kernel,
        out_type=jax.ShapeDtypeStruct((NC, NPAD, 128), _f32),
        mesh=mesh,
        scratch_types=[
            [pltpu.VMEM((K,), jnp.int32) for _ in range(NBUF)],
            [pltpu.VMEM((K,), jnp.int32) for _ in range(NBUF)],
            [pltpu.VMEM((K, 128), _f32) for _ in range(NBUF)],
            pltpu.VMEM((ZR, 128), _f32),
            pltpu.VMEM_SHARED((NPAD, 128), _f32),
            [pltpu.SemaphoreType.DMA for _ in range(NBUF)],
        ],
    )
    def k(h_hbm, src_hbm, dst_hbm, out_hbm, srcb, dstb, rowsb, zbuf, acc, gsems):
        c = lax.axis_index("c")
        s = lax.axis_index("s")
        w = c * NS + s
        ebase = w * eps
        zv = jnp.zeros((16,), _f32)
        for r in range(ZR):
            for j in range(128 // 16):
                zbuf[r, pl.ds(j * 16, 16)] = zv
        row0 = s * RPS
        for i in range(RPS // ZR):
            pltpu.sync_copy(zbuf, acc.at[pl.ds(row0 + i * ZR, ZR)])
        plsc.subcore_barrier()

        def feed(j, b):
            pltpu.sync_copy(src_hbm.at[pl.ds(ebase + j * K, K)], srcb[b])
            pltpu.sync_copy(dst_hbm.at[pl.ds(ebase + j * K, K)], dstb[b])
            pltpu.async_copy(h_hbm.at[srcb[b]], rowsb[b], gsems[b])

        def drain(b):
            pltpu.make_async_copy(h_hbm.at[srcb[b]], rowsb[b], gsems[b]).wait()
            pltpu.sync_copy(rowsb[b], acc.at[dstb[b]], add=True)

        for b in range(NBUF - 1):
            feed(b, b)

        nq = (niter - (NBUF - 1)) // NBUF

        def quad(q, carry):
            j = q * NBUF
            for b in range(NBUF):
                feed(j + b + NBUF - 1, (b + NBUF - 1) % NBUF)
                drain(b)
            return carry

        lax.fori_loop(0, nq, quad, 0)
        base = nq * NBUF
        nexti = base + NBUF - 1
        for t in range(niter - base):
            bb = t % NBUF
            if nexti <= niter - 1:
                feed(nexti, (NBUF - 1 + t) % NBUF)
                nexti += 1
            drain(bb)

        plsc.subcore_barrier()
        pltpu.sync_copy(acc.at[pl.ds(row0, RPS)], out_hbm.at[c, pl.ds(row0, RPS)])

    return k(h_tab, src_flat, dst_flat)


def _onehot(batch_blk):
    iota = lax.broadcasted_iota(jnp.int32, (R, G), 1)
    return (iota == batch_blk).astype(_f32)


def _dot(a, b):
    return jnp.dot(a, b, preferred_element_type=_f32,
                   precision=lax.Precision.DEFAULT)


def _dot_exact(a, b):
    return jnp.dot(a, b, preferred_element_type=_f32,
                   precision=lax.Precision.HIGHEST)


def _dotT(a, b):
    return lax.dot_general(a, b, (((0,), (0,)), ((), ())),
                           preferred_element_type=_f32,
                           precision=lax.Precision.HIGHEST)


def _tc_combine(parts, h, WrT, WrootT, br, batch2, dout, concat, h_split,
                with_counts):
    din_a = 256 if concat else 128
    din_h = 256 if h_split else 128

    def body(*refs):
        if with_counts:
            (p_ref, h_ref, wrt_ref, wroott_ref, br_ref, b_ref,
             out_ref, gs_ref, cnt_ref) = refs
        else:
            (p_ref, h_ref, wrt_ref, wroott_ref, br_ref, b_ref,
             out_ref, gs_ref) = refs
        i = pl.program_id(0)
        p = p_ref[...]
        wrt = wrt_ref[...]
        if concat:
            agg_mm = _dot(p[0], wrt[:128]) + _dot(p[1], wrt[128:])
        else:
            agg_mm = _dot(p[0] + p[1], wrt)
        wroott = wroott_ref[...]
        if h_split:
            hv = h_ref[...]
            h_mm = _dot(hv[0], wroott[:128]) + _dot(hv[1], wroott[128:])
        else:
            h_mm = _dot(h_ref[...], wroott)
        hp = jnp.maximum(agg_mm + h_mm + br_ref[...], 0.0)
        out_ref[...] = hp
        oh = _onehot(b_ref[...])

        @pl.when(i == 0)
        def _():
            gs_ref[...] = jnp.zeros_like(gs_ref)
            if with_counts:
                cnt_ref[...] = jnp.zeros_like(cnt_ref)

        gs_ref[...] += _dotT(oh, hp)
        if with_counts:
            cnt_ref[...] += _dotT(oh, jnp.ones((R, 1), _f32))

    out_shapes = [
        jax.ShapeDtypeStruct((N, dout), _f32),
        jax.ShapeDtypeStruct((G, dout), _f32),
    ]
    out_specs = [
        pl.BlockSpec((R, dout), lambda i: (i, 0)),
        pl.BlockSpec((G, dout), lambda i: (0, 0)),
    ]
    if with_counts:
        out_shapes.append(jax.ShapeDtypeStruct((G, 1), _f32))
        out_specs.append(pl.BlockSpec((G, 1), lambda i: (0, 0)))

    h_spec = (pl.BlockSpec((2, R, 128), lambda i: (0, i, 0)) if h_split
              else pl.BlockSpec((R, 128), lambda i: (i, 0)))
    return pl.pallas_call(
        body,
        grid=(NBLK,),
        in_specs=[
            pl.BlockSpec((2, R, 128), lambda i: (0, i, 0)),
            h_spec,
            pl.BlockSpec((din_a, dout), lambda i: (0, 0)),
            pl.BlockSpec((din_h, dout), lambda i: (0, 0)),
            pl.BlockSpec((1, dout), lambda i: (0, 0)),
            pl.BlockSpec((R, 1), lambda i: (i, 0)),
        ],
        out_specs=out_specs,
        out_shape=out_shapes,
    )(parts, h, WrT, WrootT, br, batch2)


def _tc_gate(h_pre, gs, counts, w1T, w2T, batch2, dout, mode):
    d16 = w1T.shape[1]

    def body(hp_ref, gs_ref, cnt_ref, w1_ref, w2_ref, b_ref, out_ref):
        i = pl.program_id(0)
        cnt = jnp.maximum(cnt_ref[...], 1.0)
        mean = gs_ref[...] / cnt
        z = jnp.maximum(_dot(mean, w1_ref[...]), 0.0)
        zz = _dot(z, w2_ref[...])
        gate = 1.0 / (1.0 + jnp.exp(-zz))
        oh = _onehot(b_ref[...])
        grows = _dot_exact(oh, gate)
        out = hp_ref[...] * grows
        if mode == "plain":
            out_ref[...] = out
        elif mode == "split":
            out_ref[0] = out[:, :128]
            out_ref[1] = out[:, 128:]
        else:
            @pl.when(i == 0)
            def _():
                out_ref[...] = jnp.zeros_like(out_ref)

            out_ref[...] += _dotT(oh, out)

    if mode == "plain":
        out_shape = jax.ShapeDtypeStruct((N, dout), _f32)
        out_spec = pl.BlockSpec((R, dout), lambda i: (i, 0))
    elif mode == "split":
        out_shape = jax.ShapeDtypeStruct((2, N, 128), _f32)
        out_spec = pl.BlockSpec((2, R, 128), lambda i: (0, i, 0))
    else:
        out_shape = jax.ShapeDtypeStruct((G, dout), _f32)
        out_spec = pl.BlockSpec((G, dout), lambda i: (0, 0))

    return pl.pallas_call(
        body,
        grid=(NBLK,),
        in_specs=[
            pl.BlockSpec((R, dout), lambda i: (i, 0)),
            pl.BlockSpec((G, dout), lambda i: (0, 0)),
            pl.BlockSpec((G, 1), lambda i: (0, 0)),
            pl.BlockSpec((dout, d16), lambda i: (0, 0)),
            pl.BlockSpec((d16, dout), lambda i: (0, 0)),
            pl.BlockSpec((R, 1), lambda i: (i, 0)),
        ],
        out_specs=out_spec,
        out_shape=out_shape,
    )(h_pre, gs, counts, w1T, w2T, batch2)


def _tc_head(gsf, counts, C1wT, C1b, C2wT, C2b, C3wT, C3b, C4wT, C4b):
    def body(gs_ref, cnt_ref, w1_ref, b1_ref, w2_ref, b2_ref, w3_ref, b3_ref,
             w4_ref, b4_ref, out_ref):
        cnt = jnp.maximum(cnt_ref[...], 1.0)
        g = gs_ref[...] / cnt
        g = jnp.maximum(_dot(g, w1_ref[...]) + b1_ref[...], 0.0)
        g = jnp.maximum(_dot(g, w2_ref[...]) + b2_ref[...], 0.0)
        g = jnp.maximum(_dot(g, w3_ref[...]) + b3_ref[...], 0.0)
        out_ref[...] = _dot(g, w4_ref[...]) + b4_ref[...]

    return pl.pallas_call(
        body,
        out_shape=jax.ShapeDtypeStruct((G, 10), _f32),
    )(gsf, counts, C1wT, C1b.reshape(1, -1), C2wT, C2b.reshape(1, -1),
      C3wT, C3b.reshape(1, -1), C4wT, C4b.reshape(1, -1))


def kernel(x, edge_index, batch, Wr1, br1, Wroot1, se1_w1, se1_w2, Wr2, br2,
           Wroot2, se2_w1, se2_w2, Wr3, br3, Wroot3, se3_w1, se3_w2, C1w, C1b,
           C2w, C2b, C3w, C3b, C4w, C4b):
    src, dst = edge_index[0], edge_index[1]
    eps_a = E // (NC * NS)
    eps_b = E // NS
    src_b = jnp.concatenate([src, src + N])
    dst_b = jnp.concatenate([dst, dst])
    batch2 = batch.reshape(N, 1)

    p1 = _seg_sum_sc(x, src, dst, eps_a)
    hp1, gs1, counts = _tc_combine(p1, x, Wr1.T, Wroot1.T, br1.reshape(1, -1),
                                   batch2, 128, concat=False, h_split=False,
                                   with_counts=True)
    h1 = _tc_gate(hp1, gs1, counts, se1_w1.T, se1_w2.T, batch2, 128, "plain")

    p2 = _seg_sum_sc(h1, src, dst, eps_a)
    hp2, gs2 = _tc_combine(p2, h1, Wr2.T, Wroot2.T, br2.reshape(1, -1),
                           batch2, 256, concat=False, h_split=False,
                           with_counts=False)
    h2s = _tc_gate(hp2, gs2, counts, se2_w1.T, se2_w2.T, batch2, 256, "split")

    p3 = _seg_sum_sc(h2s.reshape(2 * N, 128), src_b, dst_b, eps_b)
    hp3, gs3 = _tc_combine(p3, h2s, Wr3.T, Wroot3.T, br3.reshape(1, -1),
                           batch2, 512, concat=True, h_split=True,
                           with_counts=False)
    gsf = _tc_gate(hp3, gs3, counts, se3_w1.T, se3_w2.T, batch2, 512, "pool")

    return _tc_head(gsf, counts, C1w.T, C1b, C2w.T, C2b, C3w.T, C3b,
                    C4w.T, C4b)


# speedup vs baseline: 9.2041x; 1.4204x over previous
import functools

import jax
import jax.numpy as jnp
from jax import lax
from jax.experimental import pallas as pl
from jax.experimental.pallas import tpu as pltpu
from jax.experimental.pallas import tpu_sc as plsc

N = 10000
E = 320000
G = 64
NC = 2
NS = 16
K = 80
NBUF = 4
NPAD = 10240
RPS = NPAD // NS
ZR = 32
R = 1000
NBLK = N // R

_f32 = jnp.float32


def _seg_sum_sc(h_tab, src_flat, dst_flat, eps):
    niter = eps // K
    mesh = plsc.VectorSubcoreMesh(
        core_axis_name="c", subcore_axis_name="s", num_cores=NC, num_subcores=NS
    )

    @functools.partial(
        pl.kernel,
        out_type=jax.ShapeDtypeStruct((NC, NPAD, 128), _f32),
        mesh=mesh,
        scratch_types=[
            [pltpu.VMEM((K,), jnp.int32) for _ in range(NBUF)],
            [pltpu.VMEM((K,), jnp.int32) for _ in range(NBUF)],
            [pltpu.VMEM((K, 128), _f32) for _ in range(NBUF)],
            pltpu.VMEM((ZR, 128), _f32),
            pltpu.VMEM_SHARED((NPAD, 128), _f32),
            [pltpu.SemaphoreType.DMA for _ in range(NBUF)],
            [pltpu.SemaphoreType.DMA for _ in range(NBUF)],
            [pltpu.SemaphoreType.DMA for _ in range(NBUF)],
        ],
    )
    def k(h_hbm, src_hbm, dst_hbm, out_hbm, srcb, dstb, rowsb, zbuf, acc,
          isems, gsems, ssems):
        c = lax.axis_index("c")
        s = lax.axis_index("s")
        w = c * NS + s
        ebase = w * eps
        zv = jnp.zeros((16,), _f32)
        for r in range(ZR):
            for j in range(128 // 16):
                zbuf[r, pl.ds(j * 16, 16)] = zv
        row0 = s * RPS
        for i in range(RPS // ZR):
            pltpu.sync_copy(zbuf, acc.at[pl.ds(row0 + i * ZR, ZR)])
        plsc.subcore_barrier()

        def stage_a(j, b, swait):
            if swait:
                pltpu.make_async_copy(rowsb[b], acc.at[dstb[b]],
                                      ssems[b]).wait()
            pltpu.async_copy(src_hbm.at[pl.ds(ebase + j * K, K)], srcb[b],
                             isems[b])
            pltpu.async_copy(dst_hbm.at[pl.ds(ebase + j * K, K)], dstb[b],
                             isems[b])

        def stage_b(j, b):
            pltpu.make_async_copy(src_hbm.at[pl.ds(0, K)], srcb[b],
                                  isems[b]).wait()
            pltpu.make_async_copy(src_hbm.at[pl.ds(0, K)], dstb[b],
                                  isems[b]).wait()
            pltpu.async_copy(h_hbm.at[srcb[b]], rowsb[b], gsems[b])

        def stage_c(j, b):
            pltpu.make_async_copy(h_hbm.at[srcb[b]], rowsb[b], gsems[b]).wait()
            pltpu.async_copy(rowsb[b], acc.at[dstb[b]], ssems[b], add=True)

        stage_a(0, 0, False)
        stage_a(1, 1, False)
        stage_a(2, 2, False)
        stage_b(0, 0)
        stage_a(3, 3, False)
        stage_b(1, 1)
        stage_c(0, 0)

        def quad(q, carry):
            t0 = 4 * q + 1
            for dt in range(4):
                t = t0 + dt
                stage_a(t + 3, (1 + dt + 3) % NBUF, True)
                stage_b(t + 1, (1 + dt + 1) % NBUF)
                stage_c(t, (1 + dt) % NBUF)
            return carry

        nq = (niter - 5) // 4
        lax.fori_loop(0, nq, quad, 0)
        for t in range(1 + 4 * nq, niter):
            if t + 3 <= niter - 1:
                stage_a(t + 3, (t + 3) % NBUF, True)
            if t + 1 <= niter - 1:
                stage_b(t + 1, (t + 1) % NBUF)
            stage_c(t, t % NBUF)
        for j in range(niter - 4, niter):
            b = j % NBUF
            pltpu.make_async_copy(rowsb[b], acc.at[dstb[b]],
                                  ssems[b]).wait()

        plsc.subcore_barrier()
        pltpu.sync_copy(acc.at[pl.ds(row0, RPS)], out_hbm.at[c, pl.ds(row0, RPS)])

    return k(h_tab, src_flat, dst_flat)


def _onehot(batch_blk):
    iota = lax.broadcasted_iota(jnp.int32, (R, G), 1)
    return (iota == batch_blk).astype(_f32)


def _dot(a, b):
    return jnp.dot(a, b, preferred_element_type=_f32,
                   precision=lax.Precision.DEFAULT)


def _dot_exact(a, b):
    return jnp.dot(a, b, preferred_element_type=_f32,
                   precision=lax.Precision.HIGHEST)


def _dotT(a, b):
    return lax.dot_general(a, b, (((0,), (0,)), ((), ())),
                           preferred_element_type=_f32,
                           precision=lax.Precision.HIGHEST)


def _tc_combine(parts, h, WrT, WrootT, br, batch2, dout, concat, h_split,
                with_counts):
    din_a = 256 if concat else 128
    din_h = 256 if h_split else 128

    def body(*refs):
        if with_counts:
            (p_ref, h_ref, wrt_ref, wroott_ref, br_ref, b_ref,
             out_ref, gs_ref, cnt_ref) = refs
        else:
            (p_ref, h_ref, wrt_ref, wroott_ref, br_ref, b_ref,
             out_ref, gs_ref) = refs
        i = pl.program_id(0)
        p = p_ref[...]
        wrt = wrt_ref[...]
        if concat:
            agg_mm = _dot(p[0], wrt[:128]) + _dot(p[1], wrt[128:])
        else:
            agg_mm = _dot(p[0] + p[1], wrt)
        wroott = wroott_ref[...]
        if h_split:
            hv = h_ref[...]
            h_mm = _dot(hv[0], wroott[:128]) + _dot(hv[1], wroott[128:])
        else:
            h_mm = _dot(h_ref[...], wroott)
        hp = jnp.maximum(agg_mm + h_mm + br_ref[...], 0.0)
        out_ref[...] = hp
        oh = _onehot(b_ref[...])

        @pl.when(i == 0)
        def _():
            gs_ref[...] = jnp.zeros_like(gs_ref)
            if with_counts:
                cnt_ref[...] = jnp.zeros_like(cnt_ref)

        gs_ref[...] += _dotT(oh, hp)
        if with_counts:
            cnt_ref[...] += _dotT(oh, jnp.ones((R, 1), _f32))

    out_shapes = [
        jax.ShapeDtypeStruct((N, dout), _f32),
        jax.ShapeDtypeStruct((G, dout), _f32),
    ]
    out_specs = [
        pl.BlockSpec((R, dout), lambda i: (i, 0)),
        pl.BlockSpec((G, dout), lambda i: (0, 0)),
    ]
    if with_counts:
        out_shapes.append(jax.ShapeDtypeStruct((G, 1), _f32))
        out_specs.append(pl.BlockSpec((G, 1), lambda i: (0, 0)))

    h_spec = (pl.BlockSpec((2, R, 128), lambda i: (0, i, 0)) if h_split
              else pl.BlockSpec((R, 128), lambda i: (i, 0)))
    return pl.pallas_call(
        body,
        grid=(NBLK,),
        in_specs=[
            pl.BlockSpec((2, R, 128), lambda i: (0, i, 0)),
            h_spec,
            pl.BlockSpec((din_a, dout), lambda i: (0, 0)),
            pl.BlockSpec((din_h, dout), lambda i: (0, 0)),
            pl.BlockSpec((1, dout), lambda i: (0, 0)),
            pl.BlockSpec((R, 1), lambda i: (i, 0)),
        ],
        out_specs=out_specs,
        out_shape=out_shapes,
    )(parts, h, WrT, WrootT, br, batch2)


def _tc_gate(h_pre, gs, counts, w1T, w2T, batch2, dout, mode):
    d16 = w1T.shape[1]

    def body(hp_ref, gs_ref, cnt_ref, w1_ref, w2_ref, b_ref, out_ref):
        i = pl.program_id(0)
        cnt = jnp.maximum(cnt_ref[...], 1.0)
        mean = gs_ref[...] / cnt
        z = jnp.maximum(_dot(mean, w1_ref[...]), 0.0)
        zz = _dot(z, w2_ref[...])
        gate = 1.0 / (1.0 + jnp.exp(-zz))
        oh = _onehot(b_ref[...])
        grows = _dot_exact(oh, gate)
        out = hp_ref[...] * grows
        if mode == "plain":
            out_ref[...] = out
        elif mode == "split":
            out_ref[0] = out[:, :128]
            out_ref[1] = out[:, 128:]
        else:
            @pl.when(i == 0)
            def _():
                out_ref[...] = jnp.zeros_like(out_ref)

            out_ref[...] += _dotT(oh, out)

    if mode == "plain":
        out_shape = jax.ShapeDtypeStruct((N, dout), _f32)
        out_spec = pl.BlockSpec((R, dout), lambda i: (i, 0))
    elif mode == "split":
        out_shape = jax.ShapeDtypeStruct((2, N, 128), _f32)
        out_spec = pl.BlockSpec((2, R, 128), lambda i: (0, i, 0))
    else:
        out_shape = jax.ShapeDtypeStruct((G, dout), _f32)
        out_spec = pl.BlockSpec((G, dout), lambda i: (0, 0))

    return pl.pallas_call(
        body,
        grid=(NBLK,),
        in_specs=[
            pl.BlockSpec((R, dout), lambda i: (i, 0)),
            pl.BlockSpec((G, dout), lambda i: (0, 0)),
            pl.BlockSpec((G, 1), lambda i: (0, 0)),
            pl.BlockSpec((dout, d16), lambda i: (0, 0)),
            pl.BlockSpec((d16, dout), lambda i: (0, 0)),
            pl.BlockSpec((R, 1), lambda i: (i, 0)),
        ],
        out_specs=out_spec,
        out_shape=out_shape,
    )(h_pre, gs, counts, w1T, w2T, batch2)


def _tc_head(gsf, counts, C1wT, C1b, C2wT, C2b, C3wT, C3b, C4wT, C4b):
    def body(gs_ref, cnt_ref, w1_ref, b1_ref, w2_ref, b2_ref, w3_ref, b3_ref,
             w4_ref, b4_ref, out_ref):
        cnt = jnp.maximum(cnt_ref[...], 1.0)
        g = gs_ref[...] / cnt
        g = jnp.maximum(_dot(g, w1_ref[...]) + b1_ref[...], 0.0)
        g = jnp.maximum(_dot(g, w2_ref[...]) + b2_ref[...], 0.0)
        g = jnp.maximum(_dot(g, w3_ref[...]) + b3_ref[...], 0.0)
        out_ref[...] = _dot(g, w4_ref[...]) + b4_ref[...]

    return pl.pallas_call(
        body,
        out_shape=jax.ShapeDtypeStruct((G, 10), _f32),
    )(gsf, counts, C1wT, C1b.reshape(1, -1), C2wT, C2b.reshape(1, -1),
      C3wT, C3b.reshape(1, -1), C4wT, C4b.reshape(1, -1))


def kernel(x, edge_index, batch, Wr1, br1, Wroot1, se1_w1, se1_w2, Wr2, br2,
           Wroot2, se2_w1, se2_w2, Wr3, br3, Wroot3, se3_w1, se3_w2, C1w, C1b,
           C2w, C2b, C3w, C3b, C4w, C4b):
    src, dst = edge_index[0], edge_index[1]
    eps_a = E // (NC * NS)
    eps_b = E // NS
    src_b = jnp.concatenate([src, src + N])
    dst_b = jnp.concatenate([dst, dst])
    batch2 = batch.reshape(N, 1)

    p1 = _seg_sum_sc(x, src, dst, eps_a)
    hp1, gs1, counts = _tc_combine(p1, x, Wr1.T, Wroot1.T, br1.reshape(1, -1),
                                   batch2, 128, concat=False, h_split=False,
                                   with_counts=True)
    h1 = _tc_gate(hp1, gs1, counts, se1_w1.T, se1_w2.T, batch2, 128, "plain")

    p2 = _seg_sum_sc(h1, src, dst, eps_a)
    hp2, gs2 = _tc_combine(p2, h1, Wr2.T, Wroot2.T, br2.reshape(1, -1),
                           batch2, 256, concat=False, h_split=False,
                           with_counts=False)
    h2s = _tc_gate(hp2, gs2, counts, se2_w1.T, se2_w2.T, batch2, 256, "split")

    p3 = _seg_sum_sc(h2s.reshape(2 * N, 128), src_b, dst_b, eps_b)
    hp3, gs3 = _tc_combine(p3, h2s, Wr3.T, Wroot3.T, br3.reshape(1, -1),
                           batch2, 512, concat=True, h_split=True,
                           with_counts=False)
    gsf = _tc_gate(hp3, gs3, counts, se3_w1.T, se3_w2.T, batch2, 512, "pool")

    return _tc_head(gsf, counts, C1w.T, C1b, C2w.T, C2b, C3w.T, C3b,
                    C4w.T, C4b)
